# Initial kernel scaffold; baseline (speedup 1.0000x reference)
import jax
import jax.numpy as jnp
from jax.experimental import pallas as pl


def kernel(atom_xyz, atom_edges_displacement, cell, W_embed, W1, b1, W2, b2, W_out, W_gate, nodes, atom_edges, num_nodes, num_atom_edges):
    raise NotImplementedError("write your pallas kernel here")


# speedup vs baseline: 15.1713x; 15.1713x over previous
import functools
import math

import jax
import jax.numpy as jnp
import numpy as np
from jax.experimental import pallas as pl
from jax.experimental.pallas import tpu as pltpu

NUM_SPECIES = 119
MUL = 32
LMAX = 2
NUM_BASIS = 10
CUTOFF = 4.0
NUM_NEIGHBORS = 32.0

EDGE_BLK = 1600


def _edge_math_body(psrc_ref, pdst_ref, disp_ref, cell_ref, xsrc_ref,
                    W1_ref, b1_ref, W2_ref, b2_ref, msg_ref):
    psrc = psrc_ref[...]
    pdst = pdst_ref[...]
    disp_frac = disp_ref[...]
    cell = cell_ref[...]

    disp = (disp_frac[:, 0:1] * cell[0, 0][None, :]
            + disp_frac[:, 1:2] * cell[0, 1][None, :]
            + disp_frac[:, 2:3] * cell[0, 2][None, :])
    edge_vec = pdst - (psrc + disp)
    r2 = jnp.sum(edge_vec * edge_vec, axis=1, keepdims=True)
    r = jnp.sqrt(r2)
    u = edge_vec / (r + 1e-9)
    x, y, z = u[:, 0:1], u[:, 1:2], u[:, 2:3]

    c15 = math.sqrt(15.0)
    sh = [jnp.ones_like(x),
          math.sqrt(3.0) * x, math.sqrt(3.0) * y, math.sqrt(3.0) * z,
          c15 * x * y,
          c15 * y * z,
          (math.sqrt(5.0) / 2.0) * (3.0 * z * z - 1.0),
          c15 * x * z,
          (c15 / 2.0) * (x * x - y * y)]

    centers = jax.lax.broadcasted_iota(
        jnp.int32, (1, NUM_BASIS), 1).astype(jnp.float32) * (
            CUTOFF / (NUM_BASIS - 1))
    width = CUTOFF / NUM_BASIS
    g = jnp.exp(-0.5 * ((r - centers) / width) ** 2)
    basis = g / (jnp.sum(g, axis=1, keepdims=True) + 1e-9)

    h = basis @ W1_ref[...] + b1_ref[...][None, :]
    h = h * jax.nn.sigmoid(h)
    w = h @ W2_ref[...] + b2_ref[...][None, :]

    xs = xsrc_ref[...]
    wx0 = w[:, 0:MUL] * xs
    wx1 = w[:, MUL:2 * MUL] * xs
    wx2 = w[:, 2 * MUL:3 * MUL] * xs

    parts = [sh[0] * wx0]
    for d in range(1, 4):
        parts.append(sh[d] * wx1)
    for d in range(4, 9):
        parts.append(sh[d] * wx2)
    msg_ref[...] = jnp.concatenate(parts, axis=1)


def _edge_messages(psrc, pdst, disp_frac, cell, xsrc, W1, b1, W2, b2,
                   blocks_per_batch):
    E = psrc.shape[0]
    grid = (E // EDGE_BLK,)
    eb = lambda w: pl.BlockSpec((EDGE_BLK, w), lambda i: (i, 0))
    full = lambda a: pl.BlockSpec(a.shape, lambda i: (0,) * a.ndim)
    return pl.pallas_call(
        _edge_math_body,
        grid=grid,
        in_specs=[
            eb(3), eb(3), eb(3),
            pl.BlockSpec((1, 3, 3), lambda i: (i // blocks_per_batch, 0, 0)),
            eb(MUL),
            full(W1), full(b1), full(W2), full(b2),
        ],
        out_specs=eb((LMAX + 1) ** 2 * MUL),
        out_shape=jax.ShapeDtypeStruct((E, (LMAX + 1) ** 2 * MUL),
                                       jnp.float32),
    )(psrc, pdst, disp_frac, cell, xsrc, W1, b1, W2, b2)


NODE_BLK = 1000


def _out_transform_body(agg_ref, Wout_ref, Wgate_ref, out_ref):
    agg = agg_ref[...] * (1.0 / math.sqrt(NUM_NEIGHBORS))
    W_out = Wout_ref[...]
    s = agg[:, 0:MUL] @ W_out[0]
    gates = jax.nn.sigmoid(s @ Wgate_ref[...])
    g1, g2 = gates[:, :MUL], gates[:, MUL:]
    parts = [s * jax.nn.sigmoid(s)]
    for d in range(1, 4):
        parts.append(g1 * (agg[:, d * MUL:(d + 1) * MUL] @ W_out[1]))
    for d in range(4, 9):
        parts.append(g2 * (agg[:, d * MUL:(d + 1) * MUL] @ W_out[2]))
    out_ref[...] = jnp.concatenate(parts, axis=1)


def _out_transform(agg, W_out, W_gate):
    N, F = agg.shape
    return pl.pallas_call(
        _out_transform_body,
        grid=(N // NODE_BLK,),
        in_specs=[
            pl.BlockSpec((NODE_BLK, F), lambda i: (i, 0)),
            pl.BlockSpec(W_out.shape, lambda i: (0, 0, 0)),
            pl.BlockSpec(W_gate.shape, lambda i: (0, 0)),
        ],
        out_specs=pl.BlockSpec((NODE_BLK, F), lambda i: (i, 0)),
        out_shape=jax.ShapeDtypeStruct((N, F), jnp.float32),
    )(agg, W_out, W_gate)


def kernel(atom_xyz, atom_edges_displacement, cell, W_embed, W1, b1, W2, b2,
           W_out, W_gate, nodes, atom_edges, num_nodes, num_atom_edges):
    Bn, Np, _ = atom_xyz.shape
    Ep = atom_edges.shape[1]
    N = Bn * Np
    E = Bn * Ep

    offsets = jnp.cumsum(jnp.concatenate(
        [jnp.zeros((1,), dtype=num_nodes.dtype), num_nodes[:-1]]))
    edges = (atom_edges + offsets[:, None, None]).reshape(E, 2)
    src, dst = edges[:, 0], edges[:, 1]
    disp_frac = atom_edges_displacement.reshape(E, 3)
    pos = atom_xyz.reshape(N, 3)

    xfeat = W_embed[nodes.reshape(N)]
    psrc = pos[src]
    pdst = pos[dst]
    xsrc = xfeat[src]

    msg = _edge_messages(psrc, pdst, disp_frac, cell, xsrc, W1, b1, W2, b2,
                         Ep // EDGE_BLK)
    agg = jax.ops.segment_sum(msg, dst, num_segments=N)
    return _out_transform(agg, W_out, W_gate)


# speedup vs baseline: 17.0808x; 1.1259x over previous
import functools
import math

import jax
import jax.numpy as jnp
import numpy as np
from jax import lax
from jax.experimental import pallas as pl
from jax.experimental.pallas import tpu as pltpu
from jax.experimental.pallas import tpu_sc as plsc

NUM_SPECIES = 119
MUL = 32
LMAX = 2
NUM_BASIS = 10
CUTOFF = 4.0
NUM_NEIGHBORS = 32.0

EDGE_BLK = 1600


def _edge_math_body(psrc_ref, pdst_ref, disp_ref, cell_ref, xsrc_ref,
                    W1_ref, b1_ref, W2_ref, b2_ref, msg_ref):
    psrc = psrc_ref[...]
    pdst = pdst_ref[...]
    disp_frac = disp_ref[...]
    cell = cell_ref[...]

    disp = (disp_frac[:, 0:1] * cell[0, 0][None, :]
            + disp_frac[:, 1:2] * cell[0, 1][None, :]
            + disp_frac[:, 2:3] * cell[0, 2][None, :])
    edge_vec = pdst - (psrc + disp)
    r2 = jnp.sum(edge_vec * edge_vec, axis=1, keepdims=True)
    r = jnp.sqrt(r2)
    u = edge_vec / (r + 1e-9)
    x, y, z = u[:, 0:1], u[:, 1:2], u[:, 2:3]

    c15 = math.sqrt(15.0)
    sh = [jnp.ones_like(x),
          math.sqrt(3.0) * x, math.sqrt(3.0) * y, math.sqrt(3.0) * z,
          c15 * x * y,
          c15 * y * z,
          (math.sqrt(5.0) / 2.0) * (3.0 * z * z - 1.0),
          c15 * x * z,
          (c15 / 2.0) * (x * x - y * y)]

    centers = jax.lax.broadcasted_iota(
        jnp.int32, (1, NUM_BASIS), 1).astype(jnp.float32) * (
            CUTOFF / (NUM_BASIS - 1))
    width = CUTOFF / NUM_BASIS
    g = jnp.exp(-0.5 * ((r - centers) / width) ** 2)
    basis = g / (jnp.sum(g, axis=1, keepdims=True) + 1e-9)

    h = basis @ W1_ref[...] + b1_ref[...][None, :]
    h = h * jax.nn.sigmoid(h)
    w = h @ W2_ref[...] + b2_ref[...][None, :]

    xs = xsrc_ref[...]
    wx0 = w[:, 0:MUL] * xs
    wx1 = w[:, MUL:2 * MUL] * xs
    wx2 = w[:, 2 * MUL:3 * MUL] * xs

    parts = [sh[0] * wx0]
    for d in range(1, 4):
        parts.append(sh[d] * wx1)
    for d in range(4, 9):
        parts.append(sh[d] * wx2)
    msg_ref[...] = jnp.concatenate(parts, axis=1)


def _edge_messages(psrc, pdst, disp_frac, cell, xsrc, W1, b1, W2, b2,
                   blocks_per_batch):
    E = psrc.shape[0]
    grid = (E // EDGE_BLK,)
    eb = lambda w: pl.BlockSpec((EDGE_BLK, w), lambda i: (i, 0))
    full = lambda a: pl.BlockSpec(a.shape, lambda i: (0,) * a.ndim)
    return pl.pallas_call(
        _edge_math_body,
        grid=grid,
        in_specs=[
            eb(3), eb(3), eb(3),
            pl.BlockSpec((1, 3, 3), lambda i: (i // blocks_per_batch, 0, 0)),
            eb(MUL),
            full(W1), full(b1), full(W2), full(b2),
        ],
        out_specs=eb((LMAX + 1) ** 2 * MUL),
        out_shape=jax.ShapeDtypeStruct((E, (LMAX + 1) ** 2 * MUL),
                                       jnp.float32),
    )(psrc, pdst, disp_frac, cell, xsrc, W1, b1, W2, b2)


SC_CORES = 2
SC_TILES = 16
CHUNK = 80


def _sc_segment_sum(msg, dst, N):
    E, F = msg.shape
    FH = F // SC_CORES
    ept = E // SC_TILES
    nch = ept // CHUNK
    assert ept % CHUNK == 0
    Npad = ((N + 8 * SC_TILES - 1) // (8 * SC_TILES)) * (8 * SC_TILES)
    rows = Npad // SC_TILES
    dst3 = dst.reshape(SC_TILES, nch, CHUNK)
    zeros = jnp.zeros((rows, FH), dtype=jnp.float32)

    mesh = plsc.VectorSubcoreMesh(core_axis_name="c", subcore_axis_name="s")

    @functools.partial(
        pl.kernel,
        out_type=jax.ShapeDtypeStruct((Npad, F), jnp.float32),
        mesh=mesh,
        scratch_types=[
            pltpu.VMEM((nch, CHUNK), jnp.int32),
            pltpu.VMEM((CHUNK, FH), jnp.float32),
            pltpu.VMEM_SHARED((Npad, FH), jnp.float32),
        ],
        compiler_params=pltpu.CompilerParams(use_tc_tiling_on_sc=False),
    )
    def scatter_kernel(msg_hbm, dst_hbm, zeros_hbm, out_hbm,
                       idx_v, buf_v, acc_sh):
        c = lax.axis_index("c")
        s = lax.axis_index("s")
        col0 = c * FH
        pltpu.sync_copy(zeros_hbm, acc_sh.at[pl.ds(s * rows, rows)])
        pltpu.sync_copy(dst_hbm.at[s], idx_v)
        plsc.subcore_barrier()
        base = s * ept

        def body(j, carry):
            pltpu.sync_copy(
                msg_hbm.at[pl.ds(base + j * CHUNK, CHUNK), pl.ds(col0, FH)],
                buf_v)
            pltpu.sync_copy(buf_v, acc_sh.at[idx_v.at[j]], add=True)
            return carry

        lax.fori_loop(0, nch, body, 0)
        plsc.subcore_barrier()
        pltpu.sync_copy(
            acc_sh.at[pl.ds(s * rows, rows)],
            out_hbm.at[pl.ds(s * rows, rows), pl.ds(col0, FH)])

    return scatter_kernel(msg, dst3, zeros)[:N]


NODE_BLK = 1000


def _out_transform_body(agg_ref, Wout_ref, Wgate_ref, out_ref):
    agg = agg_ref[...] * (1.0 / math.sqrt(NUM_NEIGHBORS))
    W_out = Wout_ref[...]
    s = agg[:, 0:MUL] @ W_out[0]
    gates = jax.nn.sigmoid(s @ Wgate_ref[...])
    g1, g2 = gates[:, :MUL], gates[:, MUL:]
    parts = [s * jax.nn.sigmoid(s)]
    for d in range(1, 4):
        parts.append(g1 * (agg[:, d * MUL:(d + 1) * MUL] @ W_out[1]))
    for d in range(4, 9):
        parts.append(g2 * (agg[:, d * MUL:(d + 1) * MUL] @ W_out[2]))
    out_ref[...] = jnp.concatenate(parts, axis=1)


def _out_transform(agg, W_out, W_gate):
    N, F = agg.shape
    return pl.pallas_call(
        _out_transform_body,
        grid=(N // NODE_BLK,),
        in_specs=[
            pl.BlockSpec((NODE_BLK, F), lambda i: (i, 0)),
            pl.BlockSpec(W_out.shape, lambda i: (0, 0, 0)),
            pl.BlockSpec(W_gate.shape, lambda i: (0, 0)),
        ],
        out_specs=pl.BlockSpec((NODE_BLK, F), lambda i: (i, 0)),
        out_shape=jax.ShapeDtypeStruct((N, F), jnp.float32),
    )(agg, W_out, W_gate)


def kernel(atom_xyz, atom_edges_displacement, cell, W_embed, W1, b1, W2, b2,
           W_out, W_gate, nodes, atom_edges, num_nodes, num_atom_edges):
    Bn, Np, _ = atom_xyz.shape
    Ep = atom_edges.shape[1]
    N = Bn * Np
    E = Bn * Ep

    offsets = jnp.cumsum(jnp.concatenate(
        [jnp.zeros((1,), dtype=num_nodes.dtype), num_nodes[:-1]]))
    edges = (atom_edges + offsets[:, None, None]).reshape(E, 2)
    src, dst = edges[:, 0], edges[:, 1]
    disp_frac = atom_edges_displacement.reshape(E, 3)
    pos = atom_xyz.reshape(N, 3)

    xfeat = W_embed[nodes.reshape(N)]
    psrc = pos[src]
    pdst = pos[dst]
    xsrc = xfeat[src]

    msg = _edge_messages(psrc, pdst, disp_frac, cell, xsrc, W1, b1, W2, b2,
                         Ep // EDGE_BLK)
    agg = _sc_segment_sum(msg, dst, N)
    return _out_transform(agg, W_out, W_gate)


# speedup vs baseline: 27.0224x; 1.5820x over previous
import functools
import math

import jax
import jax.numpy as jnp
import numpy as np
from jax import lax
from jax.experimental import pallas as pl
from jax.experimental.pallas import tpu as pltpu
from jax.experimental.pallas import tpu_sc as plsc

NUM_SPECIES = 119
MUL = 32
LMAX = 2
NUM_BASIS = 10
CUTOFF = 4.0
NUM_NEIGHBORS = 32.0

EDGE_BLK = 1600


def _edge_math_body(gsrc_ref, gdst_ref, disp_ref, cell_ref,
                    W1_ref, b1_ref, W2_ref, b2_ref, msg_ref):
    gsrc = gsrc_ref[...]
    psrc = gsrc[:, 0:3]
    pdst = gdst_ref[...][:, 0:3]
    disp_frac = disp_ref[...]
    cell = cell_ref[...]

    disp = (disp_frac[:, 0:1] * cell[0, 0][None, :]
            + disp_frac[:, 1:2] * cell[0, 1][None, :]
            + disp_frac[:, 2:3] * cell[0, 2][None, :])
    edge_vec = pdst - (psrc + disp)
    r2 = jnp.sum(edge_vec * edge_vec, axis=1, keepdims=True)
    r = jnp.sqrt(r2)
    u = edge_vec / (r + 1e-9)
    x, y, z = u[:, 0:1], u[:, 1:2], u[:, 2:3]

    c15 = math.sqrt(15.0)
    sh = [jnp.ones_like(x),
          math.sqrt(3.0) * x, math.sqrt(3.0) * y, math.sqrt(3.0) * z,
          c15 * x * y,
          c15 * y * z,
          (math.sqrt(5.0) / 2.0) * (3.0 * z * z - 1.0),
          c15 * x * z,
          (c15 / 2.0) * (x * x - y * y)]

    centers = jax.lax.broadcasted_iota(
        jnp.int32, (1, NUM_BASIS), 1).astype(jnp.float32) * (
            CUTOFF / (NUM_BASIS - 1))
    width = CUTOFF / NUM_BASIS
    g = jnp.exp(-0.5 * ((r - centers) / width) ** 2)
    basis = g / (jnp.sum(g, axis=1, keepdims=True) + 1e-9)

    h = basis @ W1_ref[...] + b1_ref[...][None, :]
    h = h * jax.nn.sigmoid(h)
    w = h @ W2_ref[...] + b2_ref[...][None, :]

    xs = gsrc[:, 3:3 + MUL]
    wx0 = w[:, 0:MUL] * xs
    wx1 = w[:, MUL:2 * MUL] * xs
    wx2 = w[:, 2 * MUL:3 * MUL] * xs

    parts = [sh[0] * wx0]
    for d in range(1, 4):
        parts.append(sh[d] * wx1)
    for d in range(4, 9):
        parts.append(sh[d] * wx2)
    msg_ref[...] = jnp.concatenate(parts, axis=1)


def _edge_messages(gsrc, gdst, disp_frac, cell, W1, b1, W2, b2,
                   blocks_per_batch):
    E = gsrc.shape[0]
    grid = (E // EDGE_BLK,)
    eb = lambda w: pl.BlockSpec((EDGE_BLK, w), lambda i: (i, 0))
    full = lambda a: pl.BlockSpec(a.shape, lambda i: (0,) * a.ndim)
    return pl.pallas_call(
        _edge_math_body,
        grid=grid,
        in_specs=[
            eb(48), eb(16), eb(3),
            pl.BlockSpec((1, 3, 3), lambda i: (i // blocks_per_batch, 0, 0)),
            full(W1), full(b1), full(W2), full(b2),
        ],
        out_specs=eb((LMAX + 1) ** 2 * MUL),
        out_shape=jax.ShapeDtypeStruct((E, (LMAX + 1) ** 2 * MUL),
                                       jnp.float32),
    )(gsrc, gdst, disp_frac, cell, W1, b1, W2, b2)


PREP_BLK = 1000


def _prep_body(pos_ref, nodes_ref, Wemb_ref, tab48_ref, tab16_ref):
    pos = pos_ref[...]
    ids = nodes_ref[...]
    iota = jax.lax.broadcasted_iota(jnp.int32, (PREP_BLK, NUM_SPECIES), 1)
    onehot = (iota == ids).astype(jnp.float32)
    xfeat = onehot @ Wemb_ref[...]
    zpad = jnp.zeros((PREP_BLK, 13), dtype=jnp.float32)
    tab48_ref[...] = jnp.concatenate([pos, xfeat, zpad], axis=1)
    tab16_ref[...] = jnp.concatenate([pos, zpad], axis=1)


def _node_tables(pos, nodes, W_embed):
    N = pos.shape[0]
    return pl.pallas_call(
        _prep_body,
        grid=(N // PREP_BLK,),
        in_specs=[
            pl.BlockSpec((PREP_BLK, 3), lambda i: (i, 0)),
            pl.BlockSpec((PREP_BLK, 1), lambda i: (i, 0)),
            pl.BlockSpec(W_embed.shape, lambda i: (0, 0)),
        ],
        out_specs=[
            pl.BlockSpec((PREP_BLK, 48), lambda i: (i, 0)),
            pl.BlockSpec((PREP_BLK, 16), lambda i: (i, 0)),
        ],
        out_shape=[
            jax.ShapeDtypeStruct((N, 48), jnp.float32),
            jax.ShapeDtypeStruct((N, 16), jnp.float32),
        ],
    )(pos, nodes.reshape(N, 1), W_embed)


def _sc_edge_gather(tab48, tab16, src, dst):
    E = src.shape[0]
    nt = SC_CORES * SC_TILES
    ept = E // nt
    nch = ept // CHUNK
    assert ept % CHUNK == 0
    src3 = src.reshape(nt, nch, CHUNK)
    dst3 = dst.reshape(nt, nch, CHUNK)

    mesh = plsc.VectorSubcoreMesh(core_axis_name="c", subcore_axis_name="s")

    @functools.partial(
        pl.kernel,
        out_type=[
            jax.ShapeDtypeStruct((E, 48), jnp.float32),
            jax.ShapeDtypeStruct((E, 16), jnp.float32),
        ],
        mesh=mesh,
        scratch_types=[
            pltpu.VMEM((nch, CHUNK), jnp.int32),
            pltpu.VMEM((nch, CHUNK), jnp.int32),
            pltpu.VMEM((CHUNK, 48), jnp.float32),
            pltpu.VMEM((CHUNK, 16), jnp.float32),
            pltpu.SemaphoreType.DMA,
        ],
        compiler_params=pltpu.CompilerParams(use_tc_tiling_on_sc=False),
    )
    def gather_kernel(tab48_hbm, tab16_hbm, src_hbm, dst_hbm,
                      gsrc_hbm, gdst_hbm, isrc_v, idst_v, bs_v, bd_v, sem):
        c = lax.axis_index("c")
        s = lax.axis_index("s")
        w = s * SC_CORES + c
        pltpu.sync_copy(src_hbm.at[w], isrc_v)
        pltpu.sync_copy(dst_hbm.at[w], idst_v)
        base = w * ept

        def body(j, carry):
            pltpu.async_copy(tab48_hbm.at[isrc_v.at[j]], bs_v, sem).wait()
            pltpu.sync_copy(
                bs_v, gsrc_hbm.at[pl.ds(base + j * CHUNK, CHUNK), :])
            pltpu.async_copy(tab16_hbm.at[idst_v.at[j]], bd_v, sem).wait()
            pltpu.sync_copy(
                bd_v, gdst_hbm.at[pl.ds(base + j * CHUNK, CHUNK), :])
            return carry

        lax.fori_loop(0, nch, body, 0)

    return gather_kernel(tab48, tab16, src3, dst3)


SC_CORES = 2
SC_TILES = 16
CHUNK = 80


def _sc_segment_sum(msg, dst, N):
    E, F = msg.shape
    FH = F // SC_CORES
    ept = E // SC_TILES
    nch = ept // CHUNK
    assert ept % CHUNK == 0
    Npad = ((N + 8 * SC_TILES - 1) // (8 * SC_TILES)) * (8 * SC_TILES)
    rows = Npad // SC_TILES
    dst3 = dst.reshape(SC_TILES, nch, CHUNK)
    zeros = jnp.zeros((rows, FH), dtype=jnp.float32)

    mesh = plsc.VectorSubcoreMesh(core_axis_name="c", subcore_axis_name="s")

    @functools.partial(
        pl.kernel,
        out_type=jax.ShapeDtypeStruct((Npad, F), jnp.float32),
        mesh=mesh,
        scratch_types=[
            pltpu.VMEM((nch, CHUNK), jnp.int32),
            pltpu.VMEM((CHUNK, FH), jnp.float32),
            pltpu.VMEM_SHARED((Npad, FH), jnp.float32),
        ],
        compiler_params=pltpu.CompilerParams(use_tc_tiling_on_sc=False),
    )
    def scatter_kernel(msg_hbm, dst_hbm, zeros_hbm, out_hbm,
                       idx_v, buf_v, acc_sh):
        c = lax.axis_index("c")
        s = lax.axis_index("s")
        col0 = c * FH
        pltpu.sync_copy(zeros_hbm, acc_sh.at[pl.ds(s * rows, rows)])
        pltpu.sync_copy(dst_hbm.at[s], idx_v)
        plsc.subcore_barrier()
        base = s * ept

        def body(j, carry):
            pltpu.sync_copy(
                msg_hbm.at[pl.ds(base + j * CHUNK, CHUNK), pl.ds(col0, FH)],
                buf_v)
            pltpu.sync_copy(buf_v, acc_sh.at[idx_v.at[j]], add=True)
            return carry

        lax.fori_loop(0, nch, body, 0)
        plsc.subcore_barrier()
        pltpu.sync_copy(
            acc_sh.at[pl.ds(s * rows, rows)],
            out_hbm.at[pl.ds(s * rows, rows), pl.ds(col0, FH)])

    return scatter_kernel(msg, dst3, zeros)[:N]


NODE_BLK = 1000


def _out_transform_body(agg_ref, Wout_ref, Wgate_ref, out_ref):
    agg = agg_ref[...] * (1.0 / math.sqrt(NUM_NEIGHBORS))
    W_out = Wout_ref[...]
    s = agg[:, 0:MUL] @ W_out[0]
    gates = jax.nn.sigmoid(s @ Wgate_ref[...])
    g1, g2 = gates[:, :MUL], gates[:, MUL:]
    parts = [s * jax.nn.sigmoid(s)]
    for d in range(1, 4):
        parts.append(g1 * (agg[:, d * MUL:(d + 1) * MUL] @ W_out[1]))
    for d in range(4, 9):
        parts.append(g2 * (agg[:, d * MUL:(d + 1) * MUL] @ W_out[2]))
    out_ref[...] = jnp.concatenate(parts, axis=1)


def _out_transform(agg, W_out, W_gate):
    N, F = agg.shape
    return pl.pallas_call(
        _out_transform_body,
        grid=(N // NODE_BLK,),
        in_specs=[
            pl.BlockSpec((NODE_BLK, F), lambda i: (i, 0)),
            pl.BlockSpec(W_out.shape, lambda i: (0, 0, 0)),
            pl.BlockSpec(W_gate.shape, lambda i: (0, 0)),
        ],
        out_specs=pl.BlockSpec((NODE_BLK, F), lambda i: (i, 0)),
        out_shape=jax.ShapeDtypeStruct((N, F), jnp.float32),
    )(agg, W_out, W_gate)


def kernel(atom_xyz, atom_edges_displacement, cell, W_embed, W1, b1, W2, b2,
           W_out, W_gate, nodes, atom_edges, num_nodes, num_atom_edges):
    Bn, Np, _ = atom_xyz.shape
    Ep = atom_edges.shape[1]
    N = Bn * Np
    E = Bn * Ep

    offsets = jnp.cumsum(jnp.concatenate(
        [jnp.zeros((1,), dtype=num_nodes.dtype), num_nodes[:-1]]))
    edges = (atom_edges + offsets[:, None, None]).reshape(E, 2)
    src, dst = edges[:, 0], edges[:, 1]
    disp_frac = atom_edges_displacement.reshape(E, 3)
    pos = atom_xyz.reshape(N, 3)

    tab48, tab16 = _node_tables(pos, nodes, W_embed)
    gsrc, gdst = _sc_edge_gather(tab48, tab16, src, dst)
    msg = _edge_messages(gsrc, gdst, disp_frac, cell, W1, b1, W2, b2,
                         Ep // EDGE_BLK)
    agg = _sc_segment_sum(msg, dst, N)
    return _out_transform(agg, W_out, W_gate)
